# baseline (speedup 1.0000x reference)
import jax
import jax.numpy as jnp
from jax.experimental import pallas as pl
from jax.experimental.pallas import tpu as pltpu

_BLOCK_B = 2048


def _bcast_body(emb_ref, out_ref):
    out_ref[...] = jnp.broadcast_to(emb_ref[...], out_ref.shape)


def kernel(action_mask, embedding):
    batch = action_mask.shape[0]
    num_types, dim = embedding.shape
    flat = embedding.reshape(1, num_types * dim)
    out = pl.pallas_call(
        _bcast_body,
        grid=(batch // _BLOCK_B,),
        in_specs=[pl.BlockSpec((1, num_types * dim), lambda i: (0, 0))],
        out_specs=pl.BlockSpec((_BLOCK_B, num_types * dim), lambda i: (i, 0)),
        out_shape=jax.ShapeDtypeStruct((batch, num_types * dim), embedding.dtype),
        compiler_params=pltpu.CompilerParams(
            dimension_semantics=("parallel",),
        ),
    )(flat)
    return out.reshape(batch, num_types, dim)
